# Initial kernel scaffold; baseline (speedup 1.0000x reference)
import jax
import jax.numpy as jnp
from jax.experimental import pallas as pl


def kernel(word_inputs_data, batch_sizes, embedding_table):
    raise NotImplementedError("write your pallas kernel here")


# speedup vs baseline: 13.6748x; 13.6748x over previous
import functools

import jax
import jax.numpy as jnp
from jax import lax
from jax.experimental import pallas as pl
from jax.experimental.pallas import tpu as pltpu
from jax.experimental.pallas import tpu_sc as plsc

NC = 2
NS = 16
LANES = 16
NW = NC * NS

B = 4096
L = 200
D = 128
HALF = L // 2
B_PER_W = B // NW
D_VECS = D // LANES


def _cbow_kernel(words_hbm, bs_hbm, table_hbm, out_hbm,
                 idx_v, bs_v, recip_v, rows_a, rows_b, out_v,
                 sem_a, sem_b):
    wid = lax.axis_index("s") * NC + lax.axis_index("c")
    base = wid * B_PER_W

    pltpu.sync_copy(words_hbm.at[pl.ds(base, B_PER_W)], idx_v)
    pltpu.sync_copy(bs_hbm.at[pl.ds(base, B_PER_W)], bs_v)

    for c in range(B_PER_W // LANES):
        bsf = bs_v[pl.ds(c * LANES, LANES)].astype(jnp.float32)
        recip_v[pl.ds(c * LANES, LANES)] = 1.0 / bsf

    def start_gather(s, rows_ref, sem):
        for h in range(2):
            pltpu.make_async_copy(
                table_hbm.at[idx_v.at[s, h]],
                rows_ref.at[pl.ds(h * HALF, HALF)],
                sem,
            ).start()

    def wait_gather(rows_ref, sem):
        for h in range(2):
            pltpu.make_async_copy(
                table_hbm.at[idx_v.at[0, h]],
                rows_ref.at[pl.ds(h * HALF, HALF)],
                sem,
            ).wait()

    def accum_and_store(s, rows_ref):
        def body(l, accs):
            return tuple(accs[d] + rows_ref[l, pl.ds(d * LANES, LANES)]
                         for d in range(D_VECS))
        accs = lax.fori_loop(
            0, L, body,
            tuple(jnp.zeros((LANES,), jnp.float32) for _ in range(D_VECS)),
            unroll=4)
        r = plsc.load_gather(recip_v, [jnp.full((LANES,), s, dtype=jnp.int32)])
        for d in range(D_VECS):
            out_v[s, pl.ds(d * LANES, LANES)] = accs[d] * r

    start_gather(0, rows_a, sem_a)

    def pair_body(i, _):
        s0 = 2 * i
        start_gather(s0 + 1, rows_b, sem_b)
        wait_gather(rows_a, sem_a)
        accum_and_store(s0, rows_a)

        @pl.when(s0 + 2 < B_PER_W)
        def _():
            start_gather(s0 + 2, rows_a, sem_a)

        wait_gather(rows_b, sem_b)
        accum_and_store(s0 + 1, rows_b)
        return 0

    lax.fori_loop(0, B_PER_W // 2, pair_body, 0)

    pltpu.sync_copy(out_v, out_hbm.at[pl.ds(base, B_PER_W)])


@jax.jit
def _cbow(words, bs, table):
    run = pl.kernel(
        _cbow_kernel,
        out_type=jax.ShapeDtypeStruct((B, D), jnp.float32),
        mesh=plsc.VectorSubcoreMesh(core_axis_name="c", subcore_axis_name="s"),
        compiler_params=pltpu.CompilerParams(needs_layout_passes=False),
        scratch_types=[
            pltpu.VMEM((B_PER_W, 2, HALF), jnp.int32),
            pltpu.VMEM((B_PER_W,), jnp.int32),
            pltpu.VMEM((B_PER_W,), jnp.float32),
            pltpu.VMEM((L, D), jnp.float32),
            pltpu.VMEM((L, D), jnp.float32),
            pltpu.VMEM((B_PER_W, D), jnp.float32),
            pltpu.SemaphoreType.DMA,
            pltpu.SemaphoreType.DMA,
        ],
    )
    return run(words, bs, table)


def kernel(word_inputs_data, batch_sizes, embedding_table):
    words = word_inputs_data.astype(jnp.int32).reshape(B, 2, HALF)
    bs = batch_sizes.astype(jnp.int32)
    return _cbow(words, bs, embedding_table.astype(jnp.float32))
